# Initial kernel scaffold; baseline (speedup 1.0000x reference)
import jax
import jax.numpy as jnp
from jax.experimental import pallas as pl


def kernel(x, edge_index, W1, b1, W2, b2, Wt, bt, Wfc, bfc):
    raise NotImplementedError("write your pallas kernel here")


# speedup vs baseline: 5.6874x; 5.6874x over previous
import functools

import jax
import jax.numpy as jnp
from jax import lax
from jax.experimental import pallas as pl
from jax.experimental.pallas import tpu as pltpu
from jax.experimental.pallas import tpu_sc as plsc

N = 10000
E = 320000
D1 = 64
D2 = 128
OUT = 16
F_NUM = 2

NC = 2
NS = 16
NW = NC * NS
EPW = E // NW
CHUNK = 80
NCHUNK = EPW // CHUNK
RSTRIPE = 624
ZROWS = 16

@functools.cache
def _sc_mesh():
    return plsc.VectorSubcoreMesh(
        core_axis_name="c", subcore_axis_name="s", num_cores=NC, num_subcores=NS
    )


def _zero_vmem(ref, nrows, ncols):
    lanes = ncols // 16

    def body(i, _):
        r = i // lanes
        k = i % lanes
        ref[r, pl.ds(k * 16, 16)] = jnp.zeros((16,), jnp.float32)
        return 0

    lax.fori_loop(0, nrows * lanes, body, 0)


@functools.cache
def _sc_degree_kernel():
    return functools.partial(
        pl.kernel,
        out_type=jax.ShapeDtypeStruct((NC, N, 16), jnp.float32),
        mesh=_sc_mesh(),
        scratch_types=[
            pltpu.VMEM((CHUNK,), jnp.int32),
            pltpu.VMEM((CHUNK, 16), jnp.float32),
            pltpu.VMEM((ZROWS, 16), jnp.float32),
            pltpu.VMEM_SHARED((N, 16), jnp.float32),
        ],
    )(_sc_degree_body)


def _sc_degree(dst):
    return _sc_degree_kernel()(dst)


def _sc_degree_body(dst_hbm, deg_hbm, idx_v, ones_v, zbuf, acc):
    cid = lax.axis_index("c")
    sid = lax.axis_index("s")
    wid = sid * NC + cid

    def fill_ones(i, _):
        ones_v[i, :] = jnp.ones((16,), jnp.float32)
        return 0

    lax.fori_loop(0, CHUNK, fill_ones, 0)
    _zero_vmem(zbuf, ZROWS, 16)

    def zero_stripe(t, _):
        pltpu.sync_copy(zbuf, acc.at[pl.ds(sid * RSTRIPE + t * ZROWS, ZROWS)])
        return 0

    lax.fori_loop(0, RSTRIPE // ZROWS, zero_stripe, 0)

    @pl.when(sid == NS - 1)
    def _():
        lax.fori_loop(RSTRIPE // ZROWS, RSTRIPE // ZROWS + 1, zero_stripe, 0)

    plsc.subcore_barrier()

    base = wid * EPW

    def chunk(j, _):
        pltpu.sync_copy(dst_hbm.at[pl.ds(base + j * CHUNK, CHUNK)], idx_v)
        pltpu.sync_copy(ones_v, acc.at[idx_v], add=True)
        return 0

    lax.fori_loop(0, NCHUNK, chunk, 0)
    plsc.subcore_barrier()

    row0 = sid * RSTRIPE
    pltpu.sync_copy(
        acc.at[pl.ds(row0, RSTRIPE)],
        deg_hbm.at[cid, pl.ds(row0, RSTRIPE)],
    )

    @pl.when(sid == NS - 1)
    def _():
        pltpu.sync_copy(
            acc.at[pl.ds(NS * RSTRIPE, N - NS * RSTRIPE)],
            deg_hbm.at[cid, pl.ds(NS * RSTRIPE, N - NS * RSTRIPE)],
        )


@functools.cache
def _make_sc_aggregate(D):
    @functools.partial(
        pl.kernel,
        out_type=jax.ShapeDtypeStruct((NC, N, D), jnp.float32),
        mesh=_sc_mesh(),
        scratch_types=[
            pltpu.VMEM((CHUNK,), jnp.int32),
            pltpu.VMEM((CHUNK,), jnp.int32),
            pltpu.VMEM((CHUNK, D), jnp.float32),
            pltpu.VMEM((ZROWS, D), jnp.float32),
            pltpu.VMEM_SHARED((N, D), jnp.float32),
            pltpu.SemaphoreType.DMA,
        ],
    )
    def agg(y_hbm, src_hbm, dst_hbm, out_hbm, sidx, didx, rows, zbuf, acc, sem):
        cid = lax.axis_index("c")
        sid = lax.axis_index("s")
        wid = sid * NC + cid

        _zero_vmem(zbuf, ZROWS, D)

        def zero_stripe(t, _):
            pltpu.sync_copy(
                zbuf, acc.at[pl.ds(sid * RSTRIPE + t * ZROWS, ZROWS)]
            )
            return 0

        lax.fori_loop(0, RSTRIPE // ZROWS, zero_stripe, 0)

        @pl.when(sid == NS - 1)
        def _():
            lax.fori_loop(RSTRIPE // ZROWS, RSTRIPE // ZROWS + 1, zero_stripe, 0)

        plsc.subcore_barrier()

        base = wid * EPW

        def chunk(j, _):
            off = base + j * CHUNK
            pltpu.sync_copy(src_hbm.at[pl.ds(off, CHUNK)], sidx)
            pltpu.sync_copy(dst_hbm.at[pl.ds(off, CHUNK)], didx)
            pltpu.async_copy(y_hbm.at[sidx], rows, sem).wait()
            pltpu.sync_copy(rows, acc.at[didx], add=True)
            return 0

        lax.fori_loop(0, NCHUNK, chunk, 0)
        plsc.subcore_barrier()

        row0 = sid * RSTRIPE
        pltpu.sync_copy(
            acc.at[pl.ds(row0, RSTRIPE)],
            out_hbm.at[cid, pl.ds(row0, RSTRIPE)],
        )

        @pl.when(sid == NS - 1)
        def _():
            pltpu.sync_copy(
                acc.at[pl.ds(NS * RSTRIPE, N - NS * RSTRIPE)],
                out_hbm.at[cid, pl.ds(NS * RSTRIPE, N - NS * RSTRIPE)],
            )

    return agg


def _sc_agg128(y, src, dst):
    return _make_sc_aggregate(D2)(y, src, dst)


P = 2000
NB = N // P


def _scale1_body(x_ref, w1_ref, dp_ref, y1_ref, dinv_ref):
    deg = dp_ref[0, :, 0:1] + dp_ref[1, :, 0:1] + 1.0
    dinv = 1.0 / jnp.sqrt(deg)
    xw = jnp.dot(x_ref[...], w1_ref[...], preferred_element_type=jnp.float32)
    y1_ref[:, :D1] = xw * dinv
    y1_ref[:, D1:] = jnp.zeros((P, D2 - D1), jnp.float32)
    dinv_ref[...] = dinv


def _tc_scale1(x, W1, degp):
    return pl.pallas_call(
        _scale1_body,
        grid=(NB,),
        in_specs=[
            pl.BlockSpec((P, D2), lambda i: (i, 0)),
            pl.BlockSpec((D2, D1), lambda i: (0, 0)),
            pl.BlockSpec((NC, P, D2), lambda i: (0, i, 0)),
        ],
        out_specs=[
            pl.BlockSpec((P, D2), lambda i: (i, 0)),
            pl.BlockSpec((P, 1), lambda i: (i, 0)),
        ],
        out_shape=[
            jax.ShapeDtypeStruct((N, D2), jnp.float32),
            jax.ShapeDtypeStruct((N, 1), jnp.float32),
        ],
    )(x, W1, degp)


def _layer2_body(q_ref, y1_ref, dinv_ref, w2_ref, b1_ref, y2_ref):
    dinv = dinv_ref[...]
    s = q_ref[0, :, :D1] + q_ref[1, :, :D1] + y1_ref[:, :D1]
    h1 = jnp.maximum(dinv * s + b1_ref[...], 0.0)
    xw2 = jnp.dot(h1, w2_ref[...], preferred_element_type=jnp.float32)
    y2_ref[...] = xw2 * dinv


def _tc_layer2(accp1, y1, dinv, W2, b1):
    return pl.pallas_call(
        _layer2_body,
        grid=(NB,),
        in_specs=[
            pl.BlockSpec((NC, P, D2), lambda i: (0, i, 0)),
            pl.BlockSpec((P, D2), lambda i: (i, 0)),
            pl.BlockSpec((P, 1), lambda i: (i, 0)),
            pl.BlockSpec((D1, D2), lambda i: (0, 0)),
            pl.BlockSpec((1, D1), lambda i: (0, 0)),
        ],
        out_specs=pl.BlockSpec((P, D2), lambda i: (i, 0)),
        out_shape=jax.ShapeDtypeStruct((N, D2), jnp.float32),
    )(accp1, y1, dinv, W2, b1)


def _h2_body(r_ref, y2_ref, dinv_ref, b2_ref, h2_ref):
    h2_ref[...] = jnp.maximum(
        dinv_ref[...] * (r_ref[0] + r_ref[1] + y2_ref[...]) + b2_ref[...], 0.0
    )


def _tc_h2(accp2, y2, dinv, b2):
    return pl.pallas_call(
        _h2_body,
        grid=(NB,),
        in_specs=[
            pl.BlockSpec((NC, P, D2), lambda i: (0, i, 0)),
            pl.BlockSpec((P, D2), lambda i: (i, 0)),
            pl.BlockSpec((P, 1), lambda i: (i, 0)),
            pl.BlockSpec((1, D2), lambda i: (0, 0)),
        ],
        out_specs=pl.BlockSpec((P, D2), lambda i: (i, 0)),
        out_shape=jax.ShapeDtypeStruct((N, D2), jnp.float32),
    )(accp2, y2, dinv, b2)


def _conv_body(prev_ref, cur_ref, nxt_ref, a0_ref, a1_ref, a2_ref, bt_ref,
               vt_ref, deltas_ref):
    i = pl.program_id(0)
    cur = cur_ref[...]
    sh_l = jnp.concatenate([prev_ref[P - 1:P, :], cur[: P - 1, :]], axis=0)
    sh_r = jnp.concatenate([cur[1:, :], nxt_ref[0:1, :]], axis=0)
    conv = (
        jnp.dot(sh_l, a0_ref[...], preferred_element_type=jnp.float32)
        + jnp.dot(cur, a1_ref[...], preferred_element_type=jnp.float32)
        + jnp.dot(sh_r, a2_ref[...], preferred_element_type=jnp.float32)
    )
    v = jnp.maximum(conv + bt_ref[...], 0.0)
    vt_ref[0, :, 0, :] = v.T

    @pl.when(i == 0)
    def _():
        wrap = jnp.dot(prev_ref[P - 1:P, :], a0_ref[...],
                       preferred_element_type=jnp.float32)
        v0a = jnp.maximum(conv[0:1, :] + bt_ref[...] - wrap, 0.0)
        deltas_ref[0:1, :] = v0a - v[0:1, :]

    @pl.when(i == NB - 1)
    def _():
        wrap = jnp.dot(nxt_ref[0:1, :], a2_ref[...],
                       preferred_element_type=jnp.float32)
        vLb = jnp.maximum(conv[P - 1:P, :] + bt_ref[...] - wrap, 0.0)
        deltas_ref[1:2, :] = vLb - v[P - 1:P, :]


def _tc_conv(h2, A0, A1, A2, bt):
    return pl.pallas_call(
        _conv_body,
        grid=(NB,),
        in_specs=[
            pl.BlockSpec((P, D2), lambda i: ((i + NB - 1) % NB, 0)),
            pl.BlockSpec((P, D2), lambda i: (i, 0)),
            pl.BlockSpec((P, D2), lambda i: ((i + 1) % NB, 0)),
            pl.BlockSpec((D2, D2), lambda i: (0, 0)),
            pl.BlockSpec((D2, D2), lambda i: (0, 0)),
            pl.BlockSpec((D2, D2), lambda i: (0, 0)),
            pl.BlockSpec((1, D2), lambda i: (0, 0)),
        ],
        out_specs=[
            pl.BlockSpec((1, D2, 1, P), lambda i: (i, 0, 0, 0)),
            pl.BlockSpec((2, D2), lambda i: (0, 0)),
        ],
        out_shape=[
            jax.ShapeDtypeStruct((NB, D2, 1, P), jnp.float32),
            jax.ShapeDtypeStruct((2, D2), jnp.float32),
        ],
    )(h2, h2, h2, A0, A1, A2, bt)


def _fc_body(vt_ref, wf_ref, dblk_ref, bfc_ref, o_ref):
    c = pl.program_id(0)
    i = pl.program_id(1)
    vrow = vt_ref[0, 0]
    wa = wf_ref[0, 0]
    wb = wf_ref[0, 1]
    contrib = jnp.dot(vrow, wa + wb, preferred_element_type=jnp.float32)
    d0 = jnp.where(i == 0, dblk_ref[0, c], 0.0)
    dL = jnp.where(i == NB - 1, dblk_ref[1, c], 0.0)
    contrib += d0 * wf_ref[0, 0, 0:1, :] + dL * wf_ref[0, 1, P - 1:P, :]
    first = jnp.logical_and(c == 0, i == 0)

    @pl.when(first)
    def _():
        o_ref[...] = bfc_ref[...] + contrib

    @pl.when(jnp.logical_not(first))
    def _():
        o_ref[...] += contrib


def _tc_fc(v2t4, deltas, Wfc4, bfc):
    return pl.pallas_call(
        _fc_body,
        grid=(D2, NB),
        in_specs=[
            pl.BlockSpec((1, 1, 1, P), lambda c, i: (i, c, 0, 0)),
            pl.BlockSpec((1, 2, P, OUT), lambda c, i: (c, 0, i, 0)),
            pl.BlockSpec(memory_space=pltpu.SMEM),
            pl.BlockSpec((1, OUT), lambda c, i: (0, 0)),
        ],
        out_specs=pl.BlockSpec((1, OUT), lambda c, i: (0, 0)),
        out_shape=jax.ShapeDtypeStruct((1, OUT), jnp.float32),
    )(v2t4, Wfc4, deltas, bfc)


def kernel(x, edge_index, W1, b1, W2, b2, Wt, bt, Wfc, bfc):
    src = edge_index[0]
    dst = edge_index[1]

    ones = jnp.ones((N, D2), jnp.float32)
    degp = _sc_agg128(ones, dst, dst)
    y1, dinv = _tc_scale1(x, W1, degp)
    accp1 = _sc_agg128(y1, src, dst)
    y2 = _tc_layer2(accp1, y1, dinv, W2, b1.reshape(1, D1))
    accp2 = _sc_agg128(y2, src, dst)
    h2 = _tc_h2(accp2, y2, dinv, b2.reshape(1, D2))

    A0 = Wt[:, :, 0, 0].T
    A1 = Wt[:, :, 1, 0].T
    A2 = Wt[:, :, 2, 0].T
    v2t, deltas = _tc_conv(h2, A0, A1, A2, bt.reshape(1, D2))
    out = _tc_fc(v2t, deltas, Wfc.reshape(D2, F_NUM, N, OUT),
                 bfc.reshape(1, OUT))
    return out


# speedup vs baseline: 6.8505x; 1.2045x over previous
import functools

import jax
import jax.numpy as jnp
from jax import lax
from jax.experimental import pallas as pl
from jax.experimental.pallas import tpu as pltpu
from jax.experimental.pallas import tpu_sc as plsc

N = 10000
E = 320000
D1 = 64
D2 = 128
OUT = 16
F_NUM = 2

NC = 2
NS = 16
NW = NC * NS
EPW = E // NW
CHUNK = 80
NCHUNK = EPW // CHUNK
RSTRIPE = 624
ZROWS = 16

@functools.cache
def _sc_mesh():
    return plsc.VectorSubcoreMesh(
        core_axis_name="c", subcore_axis_name="s", num_cores=NC, num_subcores=NS
    )


def _zero_vmem(ref, nrows, ncols):
    lanes = ncols // 16

    def body(i, _):
        r = i // lanes
        k = i % lanes
        ref[r, pl.ds(k * 16, 16)] = jnp.zeros((16,), jnp.float32)
        return 0

    lax.fori_loop(0, nrows * lanes, body, 0)


GRP = 1
NGRP = NCHUNK // GRP


@functools.cache
def _make_sc_aggregate(D):
    @functools.partial(
        pl.kernel,
        out_type=jax.ShapeDtypeStruct((NC, N, D), jnp.float32),
        mesh=_sc_mesh(),
        scratch_types=[
            pltpu.VMEM((NCHUNK, 1, CHUNK), jnp.int32),
            pltpu.VMEM((NCHUNK, 1, CHUNK), jnp.int32),
            pltpu.VMEM((GRP * CHUNK, D), jnp.float32),
            pltpu.VMEM_SHARED((N, D), jnp.float32),
            pltpu.SemaphoreType.DMA,
        ],
    )
    def agg(y_hbm, src_hbm, dst_hbm, out_hbm, sidx, didx, rows, acc, sem):
        cid = lax.axis_index("c")
        sid = lax.axis_index("s")
        wid = sid * NC + cid

        _zero_vmem(rows, ZROWS, D)

        def zero_stripe(t, _):
            pltpu.sync_copy(
                rows.at[pl.ds(0, ZROWS)],
                acc.at[pl.ds(sid * RSTRIPE + t * ZROWS, ZROWS)],
            )
            return 0

        lax.fori_loop(0, RSTRIPE // ZROWS, zero_stripe, 0)

        @pl.when(sid == NS - 1)
        def _():
            lax.fori_loop(RSTRIPE // ZROWS, RSTRIPE // ZROWS + 1, zero_stripe, 0)

        pltpu.sync_copy(src_hbm.at[wid], sidx)
        pltpu.sync_copy(dst_hbm.at[wid], didx)
        plsc.subcore_barrier()

        def group(it, _):
            j0 = it * GRP
            descs = []
            for b in range(GRP):
                descs.append(pltpu.async_copy(
                    y_hbm.at[sidx.at[j0 + b, 0]],
                    rows.at[pl.ds(b * CHUNK, CHUNK)],
                    sem,
                ))
            for d in descs:
                d.wait()
            for b in range(GRP):
                pltpu.sync_copy(
                    rows.at[pl.ds(b * CHUNK, CHUNK)],
                    acc.at[didx.at[j0 + b, 0]],
                    add=True,
                )
            return 0

        lax.fori_loop(0, NGRP, group, 0)
        plsc.subcore_barrier()

        row0 = sid * RSTRIPE
        pltpu.sync_copy(
            acc.at[pl.ds(row0, RSTRIPE)],
            out_hbm.at[cid, pl.ds(row0, RSTRIPE)],
        )

        @pl.when(sid == NS - 1)
        def _():
            pltpu.sync_copy(
                acc.at[pl.ds(NS * RSTRIPE, N - NS * RSTRIPE)],
                out_hbm.at[cid, pl.ds(NS * RSTRIPE, N - NS * RSTRIPE)],
            )

    return agg


def _sc_agg128(y, src4, dst4):
    return _make_sc_aggregate(D2)(y, src4, dst4)


P = 2000
NB = N // P


def _scale1_body(x_ref, w1_ref, dp_ref, y1_ref, dinv_ref):
    deg = dp_ref[0, :, 0:1] + dp_ref[1, :, 0:1] + 1.0
    dinv = 1.0 / jnp.sqrt(deg)
    xw = jnp.dot(x_ref[...], w1_ref[...], preferred_element_type=jnp.float32)
    y1_ref[:, :D1] = xw * dinv
    y1_ref[:, D1:] = jnp.zeros((P, D2 - D1), jnp.float32)
    dinv_ref[...] = dinv


def _tc_scale1(x, W1, degp):
    return pl.pallas_call(
        _scale1_body,
        grid=(NB,),
        in_specs=[
            pl.BlockSpec((P, D2), lambda i: (i, 0)),
            pl.BlockSpec((D2, D1), lambda i: (0, 0)),
            pl.BlockSpec((NC, P, D2), lambda i: (0, i, 0)),
        ],
        out_specs=[
            pl.BlockSpec((P, D2), lambda i: (i, 0)),
            pl.BlockSpec((P, 1), lambda i: (i, 0)),
        ],
        out_shape=[
            jax.ShapeDtypeStruct((N, D2), jnp.float32),
            jax.ShapeDtypeStruct((N, 1), jnp.float32),
        ],
    )(x, W1, degp)


def _layer2_body(q_ref, y1_ref, dinv_ref, w2_ref, b1_ref, y2_ref):
    dinv = dinv_ref[...]
    s = q_ref[0, :, :D1] + q_ref[1, :, :D1] + y1_ref[:, :D1]
    h1 = jnp.maximum(dinv * s + b1_ref[...], 0.0)
    xw2 = jnp.dot(h1, w2_ref[...], preferred_element_type=jnp.float32)
    y2_ref[...] = xw2 * dinv


def _tc_layer2(accp1, y1, dinv, W2, b1):
    return pl.pallas_call(
        _layer2_body,
        grid=(NB,),
        in_specs=[
            pl.BlockSpec((NC, P, D2), lambda i: (0, i, 0)),
            pl.BlockSpec((P, D2), lambda i: (i, 0)),
            pl.BlockSpec((P, 1), lambda i: (i, 0)),
            pl.BlockSpec((D1, D2), lambda i: (0, 0)),
            pl.BlockSpec((1, D1), lambda i: (0, 0)),
        ],
        out_specs=pl.BlockSpec((P, D2), lambda i: (i, 0)),
        out_shape=jax.ShapeDtypeStruct((N, D2), jnp.float32),
    )(accp1, y1, dinv, W2, b1)


def _h2_body(r_ref, y2_ref, dinv_ref, b2_ref, h2_ref):
    h2_ref[...] = jnp.maximum(
        dinv_ref[...] * (r_ref[0] + r_ref[1] + y2_ref[...]) + b2_ref[...], 0.0
    )


def _tc_h2(accp2, y2, dinv, b2):
    return pl.pallas_call(
        _h2_body,
        grid=(NB,),
        in_specs=[
            pl.BlockSpec((NC, P, D2), lambda i: (0, i, 0)),
            pl.BlockSpec((P, D2), lambda i: (i, 0)),
            pl.BlockSpec((P, 1), lambda i: (i, 0)),
            pl.BlockSpec((1, D2), lambda i: (0, 0)),
        ],
        out_specs=pl.BlockSpec((P, D2), lambda i: (i, 0)),
        out_shape=jax.ShapeDtypeStruct((N, D2), jnp.float32),
    )(accp2, y2, dinv, b2)


def _conv_body(prev_ref, cur_ref, nxt_ref, a0_ref, a1_ref, a2_ref, bt_ref,
               vt_ref, deltas_ref):
    i = pl.program_id(0)
    cur = cur_ref[...]
    sh_l = jnp.concatenate([prev_ref[P - 1:P, :], cur[: P - 1, :]], axis=0)
    sh_r = jnp.concatenate([cur[1:, :], nxt_ref[0:1, :]], axis=0)
    conv = (
        jnp.dot(sh_l, a0_ref[...], preferred_element_type=jnp.float32)
        + jnp.dot(cur, a1_ref[...], preferred_element_type=jnp.float32)
        + jnp.dot(sh_r, a2_ref[...], preferred_element_type=jnp.float32)
    )
    v = jnp.maximum(conv + bt_ref[...], 0.0)
    vt_ref[0, :, 0, :] = v.T

    @pl.when(i == 0)
    def _():
        wrap = jnp.dot(prev_ref[P - 1:P, :], a0_ref[...],
                       preferred_element_type=jnp.float32)
        v0a = jnp.maximum(conv[0:1, :] + bt_ref[...] - wrap, 0.0)
        deltas_ref[0:1, :] = v0a - v[0:1, :]

    @pl.when(i == NB - 1)
    def _():
        wrap = jnp.dot(nxt_ref[0:1, :], a2_ref[...],
                       preferred_element_type=jnp.float32)
        vLb = jnp.maximum(conv[P - 1:P, :] + bt_ref[...] - wrap, 0.0)
        deltas_ref[1:2, :] = vLb - v[P - 1:P, :]


def _tc_conv(h2, A0, A1, A2, bt):
    return pl.pallas_call(
        _conv_body,
        grid=(NB,),
        in_specs=[
            pl.BlockSpec((P, D2), lambda i: ((i + NB - 1) % NB, 0)),
            pl.BlockSpec((P, D2), lambda i: (i, 0)),
            pl.BlockSpec((P, D2), lambda i: ((i + 1) % NB, 0)),
            pl.BlockSpec((D2, D2), lambda i: (0, 0)),
            pl.BlockSpec((D2, D2), lambda i: (0, 0)),
            pl.BlockSpec((D2, D2), lambda i: (0, 0)),
            pl.BlockSpec((1, D2), lambda i: (0, 0)),
        ],
        out_specs=[
            pl.BlockSpec((1, D2, 1, P), lambda i: (i, 0, 0, 0)),
            pl.BlockSpec((2, D2), lambda i: (0, 0)),
        ],
        out_shape=[
            jax.ShapeDtypeStruct((NB, D2, 1, P), jnp.float32),
            jax.ShapeDtypeStruct((2, D2), jnp.float32),
        ],
    )(h2, h2, h2, A0, A1, A2, bt)


def _fc_body(vt_ref, wf_ref, dblk_ref, bfc_ref, o_ref):
    c = pl.program_id(0)
    i = pl.program_id(1)
    vrow = vt_ref[0, 0]
    wa = wf_ref[0, 0]
    wb = wf_ref[0, 1]
    contrib = jnp.dot(vrow, wa + wb, preferred_element_type=jnp.float32)
    d0 = jnp.where(i == 0, dblk_ref[0, c], 0.0)
    dL = jnp.where(i == NB - 1, dblk_ref[1, c], 0.0)
    contrib += d0 * wf_ref[0, 0, 0:1, :] + dL * wf_ref[0, 1, P - 1:P, :]
    first = jnp.logical_and(c == 0, i == 0)

    @pl.when(first)
    def _():
        o_ref[...] = bfc_ref[...] + contrib

    @pl.when(jnp.logical_not(first))
    def _():
        o_ref[...] += contrib


def _tc_fc(v2t4, deltas, Wfc4, bfc):
    return pl.pallas_call(
        _fc_body,
        grid=(D2, NB),
        in_specs=[
            pl.BlockSpec((1, 1, 1, P), lambda c, i: (i, c, 0, 0)),
            pl.BlockSpec((1, 2, P, OUT), lambda c, i: (c, 0, i, 0)),
            pl.BlockSpec(memory_space=pltpu.SMEM),
            pl.BlockSpec((1, OUT), lambda c, i: (0, 0)),
        ],
        out_specs=pl.BlockSpec((1, OUT), lambda c, i: (0, 0)),
        out_shape=jax.ShapeDtypeStruct((1, OUT), jnp.float32),
    )(v2t4, Wfc4, deltas, bfc)


def kernel(x, edge_index, W1, b1, W2, b2, Wt, bt, Wfc, bfc):
    src4 = edge_index[0].reshape(NW, NCHUNK, 1, CHUNK)
    dst4 = edge_index[1].reshape(NW, NCHUNK, 1, CHUNK)

    ones = jnp.ones((N, D2), jnp.float32)
    degp = _sc_agg128(ones, src4, dst4)
    y1, dinv = _tc_scale1(x, W1, degp)
    accp1 = _sc_agg128(y1, src4, dst4)
    y2 = _tc_layer2(accp1, y1, dinv, W2, b1.reshape(1, D1))
    accp2 = _sc_agg128(y2, src4, dst4)
    h2 = _tc_h2(accp2, y2, dinv, b2.reshape(1, D2))

    A0 = Wt[:, :, 0, 0].T
    A1 = Wt[:, :, 1, 0].T
    A2 = Wt[:, :, 2, 0].T
    v2t, deltas = _tc_conv(h2, A0, A1, A2, bt.reshape(1, D2))
    out = _tc_fc(v2t, deltas, Wfc.reshape(D2, F_NUM, N, OUT),
                 bfc.reshape(1, OUT))
    return out


# speedup vs baseline: 7.6583x; 1.1179x over previous
import functools

import jax
import jax.numpy as jnp
from jax import lax
from jax.experimental import pallas as pl
from jax.experimental.pallas import tpu as pltpu
from jax.experimental.pallas import tpu_sc as plsc

N = 10000
E = 320000
D1 = 64
D2 = 128
OUT = 16
F_NUM = 2

NC = 2
NS = 16
NW = NC * NS
EPW = E // NW
CHUNK = 80
NCHUNK = EPW // CHUNK
RSTRIPE = 624
ZROWS = 16

@functools.cache
def _sc_mesh():
    return plsc.VectorSubcoreMesh(
        core_axis_name="c", subcore_axis_name="s", num_cores=NC, num_subcores=NS
    )


def _zero_vmem(ref, nrows, ncols):
    lanes = ncols // 16

    def body(i, _):
        r = i // lanes
        k = i % lanes
        ref[r, pl.ds(k * 16, 16)] = jnp.zeros((16,), jnp.float32)
        return 0

    lax.fori_loop(0, nrows * lanes, body, 0)


GRP = 1
NGRP = NCHUNK // GRP


@functools.cache
def _make_sc_aggregate(D):
    @functools.partial(
        pl.kernel,
        out_type=jax.ShapeDtypeStruct((NC, N, D), jnp.float32),
        mesh=_sc_mesh(),
        scratch_types=[
            pltpu.VMEM((EPW,), jnp.int32),
            pltpu.VMEM((NCHUNK, 1, CHUNK), jnp.int32),
            pltpu.VMEM((2, CHUNK, D), jnp.float32),
            pltpu.VMEM_SHARED((N, D), jnp.float32),
            pltpu.SemaphoreType.DMA,
        ],
    )
    def agg(y_hbm, src_hbm, dst_hbm, out_hbm, sidx, didx, rows, acc, sem):
        cid = lax.axis_index("c")
        sid = lax.axis_index("s")
        wid = sid * NC + cid

        _zero_vmem(rows.at[0], ZROWS, D)

        def zero_stripe(t, _):
            pltpu.sync_copy(
                rows.at[0, pl.ds(0, ZROWS)],
                acc.at[pl.ds(sid * RSTRIPE + t * ZROWS, ZROWS)],
            )
            return 0

        lax.fori_loop(0, RSTRIPE // ZROWS, zero_stripe, 0)

        @pl.when(sid == NS - 1)
        def _():
            lax.fori_loop(RSTRIPE // ZROWS, RSTRIPE // ZROWS + 1, zero_stripe, 0)

        pltpu.sync_copy(src_hbm.at[pl.ds(wid * EPW, EPW)], sidx)
        pltpu.sync_copy(dst_hbm.at[wid], didx)
        plsc.subcore_barrier()

        pltpu.async_copy(y_hbm.at[sidx.at[pl.ds(0, CHUNK)]], rows.at[0], sem)

        def chunk(j, _):
            cur = lax.rem(j, 2)
            pltpu.make_async_copy(
                y_hbm.at[sidx.at[pl.ds(j * CHUNK, CHUNK)]], rows.at[cur], sem
            ).wait()

            @pl.when(j + 1 < NCHUNK)
            def _():
                pltpu.async_copy(
                    y_hbm.at[sidx.at[pl.ds((j + 1) * CHUNK, CHUNK)]], rows.at[1 - cur], sem
                )

            pltpu.sync_copy(rows.at[cur], acc.at[didx.at[j, 0]], add=True)
            return 0

        lax.fori_loop(0, NCHUNK, chunk, 0)
        plsc.subcore_barrier()

        row0 = sid * RSTRIPE
        pltpu.sync_copy(
            acc.at[pl.ds(row0, RSTRIPE)],
            out_hbm.at[cid, pl.ds(row0, RSTRIPE)],
        )

        @pl.when(sid == NS - 1)
        def _():
            pltpu.sync_copy(
                acc.at[pl.ds(NS * RSTRIPE, N - NS * RSTRIPE)],
                out_hbm.at[cid, pl.ds(NS * RSTRIPE, N - NS * RSTRIPE)],
            )

    return agg


def _sc_agg128(y, src4, dst4):
    return _make_sc_aggregate(D2)(y, src4, dst4)


P = 2000
NB = N // P


def _scale1_body(x_ref, w1_ref, dp_ref, y1_ref, dinv_ref):
    deg = dp_ref[0, :, 0:1] + dp_ref[1, :, 0:1] + 1.0
    dinv = 1.0 / jnp.sqrt(deg)
    xw = jnp.dot(x_ref[...], w1_ref[...], preferred_element_type=jnp.float32)
    y1_ref[:, :D1] = xw * dinv
    y1_ref[:, D1:] = jnp.zeros((P, D2 - D1), jnp.float32)
    dinv_ref[...] = dinv


def _tc_scale1(x, W1, degp):
    return pl.pallas_call(
        _scale1_body,
        grid=(NB,),
        in_specs=[
            pl.BlockSpec((P, D2), lambda i: (i, 0)),
            pl.BlockSpec((D2, D1), lambda i: (0, 0)),
            pl.BlockSpec((NC, P, D2), lambda i: (0, i, 0)),
        ],
        out_specs=[
            pl.BlockSpec((P, D2), lambda i: (i, 0)),
            pl.BlockSpec((P, 1), lambda i: (i, 0)),
        ],
        out_shape=[
            jax.ShapeDtypeStruct((N, D2), jnp.float32),
            jax.ShapeDtypeStruct((N, 1), jnp.float32),
        ],
    )(x, W1, degp)


def _layer2_body(q_ref, y1_ref, dinv_ref, w2_ref, b1_ref, y2_ref):
    dinv = dinv_ref[...]
    s = q_ref[0, :, :D1] + q_ref[1, :, :D1] + y1_ref[:, :D1]
    h1 = jnp.maximum(dinv * s + b1_ref[...], 0.0)
    xw2 = jnp.dot(h1, w2_ref[...], preferred_element_type=jnp.float32)
    y2_ref[...] = xw2 * dinv


def _tc_layer2(accp1, y1, dinv, W2, b1):
    return pl.pallas_call(
        _layer2_body,
        grid=(NB,),
        in_specs=[
            pl.BlockSpec((NC, P, D2), lambda i: (0, i, 0)),
            pl.BlockSpec((P, D2), lambda i: (i, 0)),
            pl.BlockSpec((P, 1), lambda i: (i, 0)),
            pl.BlockSpec((D1, D2), lambda i: (0, 0)),
            pl.BlockSpec((1, D1), lambda i: (0, 0)),
        ],
        out_specs=pl.BlockSpec((P, D2), lambda i: (i, 0)),
        out_shape=jax.ShapeDtypeStruct((N, D2), jnp.float32),
    )(accp1, y1, dinv, W2, b1)


def _h2_body(r_ref, y2_ref, dinv_ref, b2_ref, h2_ref):
    h2_ref[...] = jnp.maximum(
        dinv_ref[...] * (r_ref[0] + r_ref[1] + y2_ref[...]) + b2_ref[...], 0.0
    )


def _tc_h2(accp2, y2, dinv, b2):
    return pl.pallas_call(
        _h2_body,
        grid=(NB,),
        in_specs=[
            pl.BlockSpec((NC, P, D2), lambda i: (0, i, 0)),
            pl.BlockSpec((P, D2), lambda i: (i, 0)),
            pl.BlockSpec((P, 1), lambda i: (i, 0)),
            pl.BlockSpec((1, D2), lambda i: (0, 0)),
        ],
        out_specs=pl.BlockSpec((P, D2), lambda i: (i, 0)),
        out_shape=jax.ShapeDtypeStruct((N, D2), jnp.float32),
    )(accp2, y2, dinv, b2)


def _conv_body(prev_ref, cur_ref, nxt_ref, a0_ref, a1_ref, a2_ref, bt_ref,
               vt_ref, deltas_ref):
    i = pl.program_id(0)
    cur = cur_ref[...]
    sh_l = jnp.concatenate([prev_ref[P - 1:P, :], cur[: P - 1, :]], axis=0)
    sh_r = jnp.concatenate([cur[1:, :], nxt_ref[0:1, :]], axis=0)
    conv = (
        jnp.dot(sh_l, a0_ref[...], preferred_element_type=jnp.float32)
        + jnp.dot(cur, a1_ref[...], preferred_element_type=jnp.float32)
        + jnp.dot(sh_r, a2_ref[...], preferred_element_type=jnp.float32)
    )
    v = jnp.maximum(conv + bt_ref[...], 0.0)
    vt_ref[0, :, 0, :] = v.T

    @pl.when(i == 0)
    def _():
        wrap = jnp.dot(prev_ref[P - 1:P, :], a0_ref[...],
                       preferred_element_type=jnp.float32)
        v0a = jnp.maximum(conv[0:1, :] + bt_ref[...] - wrap, 0.0)
        deltas_ref[0:1, :] = v0a - v[0:1, :]

    @pl.when(i == NB - 1)
    def _():
        wrap = jnp.dot(nxt_ref[0:1, :], a2_ref[...],
                       preferred_element_type=jnp.float32)
        vLb = jnp.maximum(conv[P - 1:P, :] + bt_ref[...] - wrap, 0.0)
        deltas_ref[1:2, :] = vLb - v[P - 1:P, :]


def _tc_conv(h2, A0, A1, A2, bt):
    return pl.pallas_call(
        _conv_body,
        grid=(NB,),
        in_specs=[
            pl.BlockSpec((P, D2), lambda i: ((i + NB - 1) % NB, 0)),
            pl.BlockSpec((P, D2), lambda i: (i, 0)),
            pl.BlockSpec((P, D2), lambda i: ((i + 1) % NB, 0)),
            pl.BlockSpec((D2, D2), lambda i: (0, 0)),
            pl.BlockSpec((D2, D2), lambda i: (0, 0)),
            pl.BlockSpec((D2, D2), lambda i: (0, 0)),
            pl.BlockSpec((1, D2), lambda i: (0, 0)),
        ],
        out_specs=[
            pl.BlockSpec((1, D2, 1, P), lambda i: (i, 0, 0, 0)),
            pl.BlockSpec((2, D2), lambda i: (0, 0)),
        ],
        out_shape=[
            jax.ShapeDtypeStruct((NB, D2, 1, P), jnp.float32),
            jax.ShapeDtypeStruct((2, D2), jnp.float32),
        ],
    )(h2, h2, h2, A0, A1, A2, bt)


def _fc_body(vt_ref, wf_ref, dblk_ref, bfc_ref, o_ref):
    c = pl.program_id(0)
    i = pl.program_id(1)
    vrow = vt_ref[0, 0]
    wa = wf_ref[0, 0]
    wb = wf_ref[0, 1]
    contrib = jnp.dot(vrow, wa + wb, preferred_element_type=jnp.float32)
    d0 = jnp.where(i == 0, dblk_ref[0, c], 0.0)
    dL = jnp.where(i == NB - 1, dblk_ref[1, c], 0.0)
    contrib += d0 * wf_ref[0, 0, 0:1, :] + dL * wf_ref[0, 1, P - 1:P, :]
    first = jnp.logical_and(c == 0, i == 0)

    @pl.when(first)
    def _():
        o_ref[...] = bfc_ref[...] + contrib

    @pl.when(jnp.logical_not(first))
    def _():
        o_ref[...] += contrib


def _tc_fc(v2t4, deltas, Wfc4, bfc):
    return pl.pallas_call(
        _fc_body,
        grid=(D2, NB),
        in_specs=[
            pl.BlockSpec((1, 1, 1, P), lambda c, i: (i, c, 0, 0)),
            pl.BlockSpec((1, 2, P, OUT), lambda c, i: (c, 0, i, 0)),
            pl.BlockSpec(memory_space=pltpu.SMEM),
            pl.BlockSpec((1, OUT), lambda c, i: (0, 0)),
        ],
        out_specs=pl.BlockSpec((1, OUT), lambda c, i: (0, 0)),
        out_shape=jax.ShapeDtypeStruct((1, OUT), jnp.float32),
    )(v2t4, Wfc4, deltas, bfc)


def kernel(x, edge_index, W1, b1, W2, b2, Wt, bt, Wfc, bfc):
    src1 = edge_index[0]
    dst4 = edge_index[1].reshape(NW, NCHUNK, 1, CHUNK)

    ones = jnp.ones((N, D2), jnp.float32)
    degp = _sc_agg128(ones, src1, dst4)
    y1, dinv = _tc_scale1(x, W1, degp)
    accp1 = _sc_agg128(y1, src1, dst4)
    y2 = _tc_layer2(accp1, y1, dinv, W2, b1.reshape(1, D1))
    accp2 = _sc_agg128(y2, src1, dst4)
    h2 = _tc_h2(accp2, y2, dinv, b2.reshape(1, D2))

    A0 = Wt[:, :, 0, 0].T
    A1 = Wt[:, :, 1, 0].T
    A2 = Wt[:, :, 2, 0].T
    v2t, deltas = _tc_conv(h2, A0, A1, A2, bt.reshape(1, D2))
    out = _tc_fc(v2t, deltas, Wfc.reshape(D2, F_NUM, N, OUT),
                 bfc.reshape(1, OUT))
    return out


# speedup vs baseline: 8.2303x; 1.0747x over previous
import functools

import jax
import jax.numpy as jnp
from jax import lax
from jax.experimental import pallas as pl
from jax.experimental.pallas import tpu as pltpu
from jax.experimental.pallas import tpu_sc as plsc

N = 10000
E = 320000
D1 = 64
D2 = 128
OUT = 16
F_NUM = 2

NC = 2
NS = 16
NW = NC * NS
EPW = E // NW
CHUNK = 80
NCHUNK = EPW // CHUNK
RSTRIPE = 624
ZROWS = 16

@functools.cache
def _sc_mesh():
    return plsc.VectorSubcoreMesh(
        core_axis_name="c", subcore_axis_name="s", num_cores=NC, num_subcores=NS
    )


def _zero_vmem(ref, nrows, ncols):
    lanes = ncols // 16

    def body(i, _):
        r = i // lanes
        k = i % lanes
        ref[r, pl.ds(k * 16, 16)] = jnp.zeros((16,), jnp.float32)
        return 0

    lax.fori_loop(0, nrows * lanes, body, 0)


GRP = 1
NGRP = NCHUNK // GRP


@functools.cache
def _sc_degree_kernel():
    return functools.partial(
        pl.kernel,
        out_type=jax.ShapeDtypeStruct((NC, N, D2), jnp.float32),
        mesh=_sc_mesh(),
        scratch_types=[
            pltpu.VMEM((NCHUNK, CHUNK), jnp.int32),
            pltpu.VMEM((CHUNK, D2), jnp.float32),
            pltpu.VMEM_SHARED((N, D2), jnp.float32),
        ],
    )(_sc_degree_body)


def _sc_degree(dst2):
    return _sc_degree_kernel()(dst2)


def _sc_degree_body(dst_hbm, deg_hbm, didx, ones_v, acc):
    cid = lax.axis_index("c")
    sid = lax.axis_index("s")
    wid = sid * NC + cid

    def fill(val):
        def body(i, _):
            r = i // (D2 // 16)
            k = i % (D2 // 16)
            ones_v[r, pl.ds(k * 16, 16)] = jnp.full((16,), val, jnp.float32)
            return 0

        lax.fori_loop(0, CHUNK * (D2 // 16), body, 0)

    def zero_stripe(t, _):
        pltpu.sync_copy(
            ones_v.at[pl.ds(0, ZROWS)],
            acc.at[pl.ds(sid * RSTRIPE + t * ZROWS, ZROWS)],
        )
        return 0

    fill(0.0)
    lax.fori_loop(0, RSTRIPE // ZROWS, zero_stripe, 0)

    @pl.when(sid == NS - 1)
    def _():
        lax.fori_loop(RSTRIPE // ZROWS, RSTRIPE // ZROWS + 1, zero_stripe, 0)

    fill(1.0)
    pltpu.sync_copy(dst_hbm.at[wid], didx)
    plsc.subcore_barrier()

    def chunk(j, _):
        pltpu.sync_copy(ones_v, acc.at[didx.at[j]], add=True)
        return 0

    lax.fori_loop(0, NCHUNK, chunk, 0)
    plsc.subcore_barrier()

    row0 = sid * RSTRIPE
    pltpu.sync_copy(
        acc.at[pl.ds(row0, RSTRIPE)],
        deg_hbm.at[cid, pl.ds(row0, RSTRIPE)],
    )

    @pl.when(sid == NS - 1)
    def _():
        pltpu.sync_copy(
            acc.at[pl.ds(NS * RSTRIPE, N - NS * RSTRIPE)],
            deg_hbm.at[cid, pl.ds(NS * RSTRIPE, N - NS * RSTRIPE)],
        )


@functools.cache
def _make_sc_aggregate(D):
    @functools.partial(
        pl.kernel,
        out_type=jax.ShapeDtypeStruct((NC, N, D), jnp.float32),
        mesh=_sc_mesh(),
        scratch_types=[
            pltpu.VMEM((EPW,), jnp.int32),
            pltpu.VMEM((NCHUNK, CHUNK), jnp.int32),
            pltpu.VMEM((2, CHUNK, D), jnp.float32),
            pltpu.VMEM_SHARED((N, D), jnp.float32),
            pltpu.SemaphoreType.DMA,
        ],
    )
    def agg(y_hbm, src_hbm, dst_hbm, out_hbm, sidx, didx, rows, acc, sem):
        cid = lax.axis_index("c")
        sid = lax.axis_index("s")
        wid = sid * NC + cid

        _zero_vmem(rows.at[0], ZROWS, D)

        def zero_stripe(t, _):
            pltpu.sync_copy(
                rows.at[0, pl.ds(0, ZROWS)],
                acc.at[pl.ds(sid * RSTRIPE + t * ZROWS, ZROWS)],
            )
            return 0

        lax.fori_loop(0, RSTRIPE // ZROWS, zero_stripe, 0)

        @pl.when(sid == NS - 1)
        def _():
            lax.fori_loop(RSTRIPE // ZROWS, RSTRIPE // ZROWS + 1, zero_stripe, 0)

        pltpu.sync_copy(src_hbm.at[pl.ds(wid * EPW, EPW)], sidx)
        pltpu.sync_copy(dst_hbm.at[wid], didx)
        plsc.subcore_barrier()

        pltpu.async_copy(y_hbm.at[sidx.at[pl.ds(0, CHUNK)]], rows.at[0], sem)

        def chunk(j, _):
            cur = lax.rem(j, 2)
            pltpu.make_async_copy(
                y_hbm.at[sidx.at[pl.ds(j * CHUNK, CHUNK)]], rows.at[cur], sem
            ).wait()

            @pl.when(j + 1 < NCHUNK)
            def _():
                pltpu.async_copy(
                    y_hbm.at[sidx.at[pl.ds((j + 1) * CHUNK, CHUNK)]], rows.at[1 - cur], sem
                )

            pltpu.sync_copy(rows.at[cur], acc.at[didx.at[j]], add=True)
            return 0

        lax.fori_loop(0, NCHUNK, chunk, 0)
        plsc.subcore_barrier()

        row0 = sid * RSTRIPE
        pltpu.sync_copy(
            acc.at[pl.ds(row0, RSTRIPE)],
            out_hbm.at[cid, pl.ds(row0, RSTRIPE)],
        )

        @pl.when(sid == NS - 1)
        def _():
            pltpu.sync_copy(
                acc.at[pl.ds(NS * RSTRIPE, N - NS * RSTRIPE)],
                out_hbm.at[cid, pl.ds(NS * RSTRIPE, N - NS * RSTRIPE)],
            )

    return agg


def _sc_agg128(y, src4, dst4):
    return _make_sc_aggregate(D2)(y, src4, dst4)


P = 2000
NB = N // P


def _scale1_body(x_ref, w1_ref, dp_ref, y1_ref, dinv_ref):
    deg = dp_ref[0, :, 0:1] + dp_ref[1, :, 0:1] + 1.0
    dinv = 1.0 / jnp.sqrt(deg)
    xw = jnp.dot(x_ref[...], w1_ref[...], preferred_element_type=jnp.float32)
    y1_ref[:, :D1] = xw * dinv
    y1_ref[:, D1:] = jnp.zeros((P, D2 - D1), jnp.float32)
    dinv_ref[...] = dinv


def _tc_scale1(x, W1, degp):
    return pl.pallas_call(
        _scale1_body,
        grid=(NB,),
        in_specs=[
            pl.BlockSpec((P, D2), lambda i: (i, 0)),
            pl.BlockSpec((D2, D1), lambda i: (0, 0)),
            pl.BlockSpec((NC, P, D2), lambda i: (0, i, 0)),
        ],
        out_specs=[
            pl.BlockSpec((P, D2), lambda i: (i, 0)),
            pl.BlockSpec((P, 1), lambda i: (i, 0)),
        ],
        out_shape=[
            jax.ShapeDtypeStruct((N, D2), jnp.float32),
            jax.ShapeDtypeStruct((N, 1), jnp.float32),
        ],
    )(x, W1, degp)


def _layer2_body(q_ref, y1_ref, dinv_ref, w2_ref, b1_ref, y2_ref):
    dinv = dinv_ref[...]
    s = q_ref[0, :, :D1] + q_ref[1, :, :D1] + y1_ref[:, :D1]
    h1 = jnp.maximum(dinv * s + b1_ref[...], 0.0)
    xw2 = jnp.dot(h1, w2_ref[...], preferred_element_type=jnp.float32)
    y2_ref[...] = xw2 * dinv


def _tc_layer2(accp1, y1, dinv, W2, b1):
    return pl.pallas_call(
        _layer2_body,
        grid=(NB,),
        in_specs=[
            pl.BlockSpec((NC, P, D2), lambda i: (0, i, 0)),
            pl.BlockSpec((P, D2), lambda i: (i, 0)),
            pl.BlockSpec((P, 1), lambda i: (i, 0)),
            pl.BlockSpec((D1, D2), lambda i: (0, 0)),
            pl.BlockSpec((1, D1), lambda i: (0, 0)),
        ],
        out_specs=pl.BlockSpec((P, D2), lambda i: (i, 0)),
        out_shape=jax.ShapeDtypeStruct((N, D2), jnp.float32),
    )(accp1, y1, dinv, W2, b1)


def _h2_body(r_ref, y2_ref, dinv_ref, b2_ref, h2_ref):
    h2_ref[...] = jnp.maximum(
        dinv_ref[...] * (r_ref[0] + r_ref[1] + y2_ref[...]) + b2_ref[...], 0.0
    )


def _tc_h2(accp2, y2, dinv, b2):
    return pl.pallas_call(
        _h2_body,
        grid=(NB,),
        in_specs=[
            pl.BlockSpec((NC, P, D2), lambda i: (0, i, 0)),
            pl.BlockSpec((P, D2), lambda i: (i, 0)),
            pl.BlockSpec((P, 1), lambda i: (i, 0)),
            pl.BlockSpec((1, D2), lambda i: (0, 0)),
        ],
        out_specs=pl.BlockSpec((P, D2), lambda i: (i, 0)),
        out_shape=jax.ShapeDtypeStruct((N, D2), jnp.float32),
    )(accp2, y2, dinv, b2)


def _conv_body(prev_ref, cur_ref, nxt_ref, a0_ref, a1_ref, a2_ref, bt_ref,
               vt_ref, deltas_ref):
    i = pl.program_id(0)
    cur = cur_ref[...]
    sh_l = jnp.concatenate([prev_ref[P - 1:P, :], cur[: P - 1, :]], axis=0)
    sh_r = jnp.concatenate([cur[1:, :], nxt_ref[0:1, :]], axis=0)
    conv = (
        jnp.dot(sh_l, a0_ref[...], preferred_element_type=jnp.float32)
        + jnp.dot(cur, a1_ref[...], preferred_element_type=jnp.float32)
        + jnp.dot(sh_r, a2_ref[...], preferred_element_type=jnp.float32)
    )
    v = jnp.maximum(conv + bt_ref[...], 0.0)
    vt_ref[0, :, 0, :] = v.T

    @pl.when(i == 0)
    def _():
        wrap = jnp.dot(prev_ref[P - 1:P, :], a0_ref[...],
                       preferred_element_type=jnp.float32)
        v0a = jnp.maximum(conv[0:1, :] + bt_ref[...] - wrap, 0.0)
        deltas_ref[0:1, :] = v0a - v[0:1, :]

    @pl.when(i == NB - 1)
    def _():
        wrap = jnp.dot(nxt_ref[0:1, :], a2_ref[...],
                       preferred_element_type=jnp.float32)
        vLb = jnp.maximum(conv[P - 1:P, :] + bt_ref[...] - wrap, 0.0)
        deltas_ref[1:2, :] = vLb - v[P - 1:P, :]


def _tc_conv(h2, A0, A1, A2, bt):
    return pl.pallas_call(
        _conv_body,
        grid=(NB,),
        in_specs=[
            pl.BlockSpec((P, D2), lambda i: ((i + NB - 1) % NB, 0)),
            pl.BlockSpec((P, D2), lambda i: (i, 0)),
            pl.BlockSpec((P, D2), lambda i: ((i + 1) % NB, 0)),
            pl.BlockSpec((D2, D2), lambda i: (0, 0)),
            pl.BlockSpec((D2, D2), lambda i: (0, 0)),
            pl.BlockSpec((D2, D2), lambda i: (0, 0)),
            pl.BlockSpec((1, D2), lambda i: (0, 0)),
        ],
        out_specs=[
            pl.BlockSpec((1, D2, 1, P), lambda i: (i, 0, 0, 0)),
            pl.BlockSpec((2, D2), lambda i: (0, 0)),
        ],
        out_shape=[
            jax.ShapeDtypeStruct((NB, D2, 1, P), jnp.float32),
            jax.ShapeDtypeStruct((2, D2), jnp.float32),
        ],
    )(h2, h2, h2, A0, A1, A2, bt)


def _fc_body(vt_ref, wf_ref, dblk_ref, bfc_ref, o_ref):
    c = pl.program_id(0)
    i = pl.program_id(1)
    vrow = vt_ref[0, 0]
    wa = wf_ref[0, 0]
    wb = wf_ref[0, 1]
    contrib = jnp.dot(vrow, wa + wb, preferred_element_type=jnp.float32)
    d0 = jnp.where(i == 0, dblk_ref[0, c], 0.0)
    dL = jnp.where(i == NB - 1, dblk_ref[1, c], 0.0)
    contrib += d0 * wf_ref[0, 0, 0:1, :] + dL * wf_ref[0, 1, P - 1:P, :]
    first = jnp.logical_and(c == 0, i == 0)

    @pl.when(first)
    def _():
        o_ref[...] = bfc_ref[...] + contrib

    @pl.when(jnp.logical_not(first))
    def _():
        o_ref[...] += contrib


def _tc_fc(v2t4, deltas, Wfc4, bfc):
    return pl.pallas_call(
        _fc_body,
        grid=(D2, NB),
        in_specs=[
            pl.BlockSpec((1, 1, 1, P), lambda c, i: (i, c, 0, 0)),
            pl.BlockSpec((1, 2, P, OUT), lambda c, i: (c, 0, i, 0)),
            pl.BlockSpec(memory_space=pltpu.SMEM),
            pl.BlockSpec((1, OUT), lambda c, i: (0, 0)),
        ],
        out_specs=pl.BlockSpec((1, OUT), lambda c, i: (0, 0)),
        out_shape=jax.ShapeDtypeStruct((1, OUT), jnp.float32),
    )(v2t4, Wfc4, deltas, bfc)


def kernel(x, edge_index, W1, b1, W2, b2, Wt, bt, Wfc, bfc):
    src1 = edge_index[0]
    dst2 = edge_index[1].reshape(NW, NCHUNK, CHUNK)

    degp = _sc_degree(dst2)
    y1, dinv = _tc_scale1(x, W1, degp)
    accp1 = _sc_agg128(y1, src1, dst2)
    y2 = _tc_layer2(accp1, y1, dinv, W2, b1.reshape(1, D1))
    accp2 = _sc_agg128(y2, src1, dst2)
    h2 = _tc_h2(accp2, y2, dinv, b2.reshape(1, D2))

    A0 = Wt[:, :, 0, 0].T
    A1 = Wt[:, :, 1, 0].T
    A2 = Wt[:, :, 2, 0].T
    v2t, deltas = _tc_conv(h2, A0, A1, A2, bt.reshape(1, D2))
    out = _tc_fc(v2t, deltas, Wfc.reshape(D2, F_NUM, N, OUT),
                 bfc.reshape(1, OUT))
    return out
